# baseline (speedup 1.0000x reference)
import jax
import jax.numpy as jnp
from jax import lax
from jax.experimental import pallas as pl
from jax.experimental.pallas import tpu as pltpu
from jax.experimental.pallas import tpu_sc as plsc

_NC = 2
_NS = 16
_NW = _NC * _NS
_BLK = 2048


def _sc_gather_pair(u_emb, m_emb, u, m):
    b = u.shape[0]
    d = u_emb.shape[1]
    b_half = b // _NC
    chunk = 2048
    n_chunks = b_half // chunk
    mesh = plsc.ScalarSubcoreMesh(axis_name="core", num_cores=_NC)
    out_t = jax.ShapeDtypeStruct((b, d), jnp.float32)

    @pl.kernel(
        out_type=(out_t, out_t),
        mesh=mesh,
        scratch_types=[
            pltpu.SMEM((chunk,), jnp.int32),
            pltpu.SMEM((chunk,), jnp.int32),
            pltpu.SemaphoreType.DMA,
            pltpu.SemaphoreType.DMA,
            pltpu.SemaphoreType.DMA,
        ],
    )
    def gather_kernel(utab_hbm, mtab_hbm, uidx_hbm, midx_hbm,
                      ue_hbm, me_hbm, us_s, ms_s, sem_i, sem_u, sem_m):
        cid = lax.axis_index("core")
        base = cid * b_half
        for c in range(n_chunks):
            off = base + c * chunk
            pltpu.async_copy(uidx_hbm.at[pl.ds(off, chunk)], us_s, sem_i).wait()
            pltpu.async_copy(midx_hbm.at[pl.ds(off, chunk)], ms_s, sem_i).wait()

            @pl.loop(0, chunk)
            def _(i):
                pltpu.make_async_copy(
                    utab_hbm.at[pl.ds(us_s[i], 1)],
                    ue_hbm.at[pl.ds(off + i, 1)], sem_u).start()
                pltpu.make_async_copy(
                    mtab_hbm.at[pl.ds(ms_s[i], 1)],
                    me_hbm.at[pl.ds(off + i, 1)], sem_m).start()

            @pl.loop(0, chunk)
            def _(i):
                pltpu.make_async_copy(
                    utab_hbm.at[pl.ds(0, 1)],
                    ue_hbm.at[pl.ds(base, 1)], sem_u).wait()
                pltpu.make_async_copy(
                    mtab_hbm.at[pl.ds(0, 1)],
                    me_hbm.at[pl.ds(base, 1)], sem_m).wait()

    return gather_kernel(u_emb, m_emb, u, m)


def _mlp_body(ue_ref, me_ref, w1a_ref, w1b_ref, b1_ref, w2_ref, b2_ref, out_ref):
    h = jnp.dot(ue_ref[...], w1a_ref[...], preferred_element_type=jnp.float32)
    h = h + jnp.dot(me_ref[...], w1b_ref[...], preferred_element_type=jnp.float32)
    h = h + b1_ref[...]
    h = jnp.maximum(h, 0.0)
    out_ref[...] = (
        jnp.dot(h, w2_ref[...], preferred_element_type=jnp.float32) + b2_ref[...]
    )


def _tc_mlp(ue, me, W1, b1, W2, b2):
    b = ue.shape[0]
    d = ue.shape[1]
    nh = W1.shape[0]
    w1a = W1[:, :d].T
    w1b = W1[:, d:].T
    b1r = b1.reshape(1, nh)
    w2 = W2.T
    b2r = b2.reshape(1, 1)
    grid = (b // _BLK,)
    return pl.pallas_call(
        _mlp_body,
        grid=grid,
        in_specs=[
            pl.BlockSpec((_BLK, d), lambda i: (i, 0)),
            pl.BlockSpec((_BLK, d), lambda i: (i, 0)),
            pl.BlockSpec((d, nh), lambda i: (0, 0)),
            pl.BlockSpec((d, nh), lambda i: (0, 0)),
            pl.BlockSpec((1, nh), lambda i: (0, 0)),
            pl.BlockSpec((nh, 1), lambda i: (0, 0)),
            pl.BlockSpec((1, 1), lambda i: (0, 0)),
        ],
        out_specs=pl.BlockSpec((_BLK, 1), lambda i: (i, 0)),
        out_shape=jax.ShapeDtypeStruct((b, 1), jnp.float32),
    )(ue, me, w1a, w1b, b1r, w2, b2r)


def kernel(u, m, u_emb, m_emb, W1, b1, W2, b2):
    ue, me = _sc_gather_pair(u_emb, m_emb, u.astype(jnp.int32), m.astype(jnp.int32))
    return _tc_mlp(ue, me, W1, b1, W2, b2)
